# Initial kernel scaffold; baseline (speedup 1.0000x reference)
import jax
import jax.numpy as jnp
from jax.experimental import pallas as pl


def kernel(x, edge_index, l1_w_self, l1_w_rel, l1_bias, l1_wq, l1_wk, l1_watt, l2_w_self, l2_w_rel, l2_bias, l2_wq, l2_wk, l2_watt, mlp_w1, mlp_b1, mlp_w2, mlp_b2):
    raise NotImplementedError("write your pallas kernel here")


# speedup vs baseline: 4.9825x; 4.9825x over previous
import functools

import jax
import jax.numpy as jnp
from jax import lax
from jax.experimental import pallas as pl
from jax.experimental.pallas import tpu as pltpu
from jax.experimental.pallas import tpu_sc as plsc

N = 10000
E = 320000
D = 128
ATT = 64

NC = 2
NS = 16
CHUNK = 80

BN = 1000
GRID = N // BN

_f32 = jnp.float32


EDGES_PER_WORKER = E // (NC * NS)
NSTEPS = EDGES_PER_WORKER // CHUNK
ROWS_PER_SUB = N // NS
ZROWS = 125


def _spmm_body(table_hbm, src_hbm, dst_hbm, out_hbm,
               src_v, dst_v, rows_v, zbuf_v, acc_sh, sem):
    c = lax.axis_index("c")
    s = lax.axis_index("s")

    zeros16 = jnp.zeros((16,), _f32)

    def _zb(i, carry):
        r = i // (D // 16)
        j = i % (D // 16)
        zbuf_v[r, pl.ds(j * 16, 16)] = zeros16
        return carry

    lax.fori_loop(0, ZROWS * (D // 16), _zb, 0)

    def _zc(r, carry):
        pltpu.sync_copy(
            zbuf_v, acc_sh.at[pl.ds(s * ROWS_PER_SUB + r * ZROWS, ZROWS)])
        return carry

    lax.fori_loop(0, ROWS_PER_SUB // ZROWS, _zc, 0)
    plsc.subcore_barrier()

    base = (s * NC + c) * EDGES_PER_WORKER

    def _step(k, carry):
        e0 = base + k * CHUNK
        pltpu.sync_copy(src_hbm.at[pl.ds(e0, CHUNK)], src_v)
        pltpu.sync_copy(dst_hbm.at[pl.ds(e0, CHUNK)], dst_v)
        pltpu.async_copy(table_hbm.at[src_v], rows_v, sem).wait()
        pltpu.sync_copy(rows_v, acc_sh.at[dst_v], add=True)
        return carry

    lax.fori_loop(0, NSTEPS, _step, 0)
    plsc.subcore_barrier()

    @pl.when(s < 10)
    def _():
        pltpu.sync_copy(
            acc_sh.at[pl.ds(s * 1000, 1000)],
            out_hbm.at[pl.ds(c * N + s * 1000, 1000)])


@functools.lru_cache(maxsize=None)
def _spmm_fn():
    return pl.kernel(
        _spmm_body,
        out_type=jax.ShapeDtypeStruct((2 * N, D), _f32),
        mesh=plsc.VectorSubcoreMesh(core_axis_name="c", subcore_axis_name="s"),
        scratch_types=[
            pltpu.VMEM((CHUNK,), jnp.int32),
            pltpu.VMEM((CHUNK,), jnp.int32),
            pltpu.VMEM((CHUNK, D), _f32),
            pltpu.VMEM((ZROWS, D), _f32),
            pltpu.VMEM_SHARED((N, D), _f32),
            pltpu.SemaphoreType.DMA,
        ],
    )


def _spmm(table, src, dst):
    return _spmm_fn()(table, src, dst)


def _mm2_body(x_ref, ws_ref, wr_ref, sf_ref, nb_ref):
    xb = x_ref[...]
    sf_ref[...] = jnp.dot(xb, ws_ref[...], preferred_element_type=_f32)
    nb_ref[...] = jnp.dot(xb, wr_ref[...], preferred_element_type=_f32)


def _tc_a(x, w_self, w_rel):
    return pl.pallas_call(
        _mm2_body,
        grid=(GRID,),
        in_specs=[
            pl.BlockSpec((BN, D), lambda i: (i, 0)),
            pl.BlockSpec((D, D), lambda i: (0, 0)),
            pl.BlockSpec((D, D), lambda i: (0, 0)),
        ],
        out_specs=[
            pl.BlockSpec((BN, D), lambda i: (i, 0)),
            pl.BlockSpec((BN, D), lambda i: (i, 0)),
        ],
        out_shape=[
            jax.ShapeDtypeStruct((N, D), _f32),
            jax.ShapeDtypeStruct((N, D), _f32),
        ],
    )(x, w_self, w_rel)


def _elu(v):
    return jnp.where(v > 0, v, jnp.exp(jnp.minimum(v, 0.0)) - 1.0)


def _attention(sf, nb, wq_ref, wk_ref, watt_ref):
    wvk = jnp.dot(wk_ref[...], watt_ref[0:ATT, :], preferred_element_type=_f32)
    wvq = jnp.dot(wq_ref[...], watt_ref[ATT:2 * ATT, :],
                  preferred_element_type=_f32)
    q = jnp.dot(sf, wvq, preferred_element_type=_f32)
    ks = jnp.dot(sf, wvk, preferred_element_type=_f32)
    kn = jnp.dot(nb, wvk, preferred_element_type=_f32)
    e0 = _elu(ks + q)
    e1 = _elu(kn + q)
    m = jnp.maximum(e0, e1)
    z0 = jnp.exp(e0 - m)
    z1 = jnp.exp(e1 - m)
    zs = z0 + z1
    return z0 / zs, z1 / zs


def _attn1_body(sf_ref, nb0_ref, nb1_ref, wq_ref, wk_ref, watt_ref, b_ref,
                ws2_ref, wr2_ref, att_ref, sf2_ref, nb2_ref):
    sf = sf_ref[...]
    nb = nb0_ref[...] + nb1_ref[...]
    a0, a1 = _attention(sf, nb, wq_ref, wk_ref, watt_ref)
    att_ref[...] = jnp.concatenate([a0, a1], axis=1)
    h = _elu(a0 * sf + a1 * nb + b_ref[...])
    sf2_ref[...] = jnp.dot(h, ws2_ref[...], preferred_element_type=_f32)
    nb2_ref[...] = jnp.dot(h, wr2_ref[...], preferred_element_type=_f32)


def _tc_b(sf, nbh, wq, wk, watt, bias, w_self2, w_rel2):
    return pl.pallas_call(
        _attn1_body,
        grid=(GRID,),
        in_specs=[
            pl.BlockSpec((BN, D), lambda i: (i, 0)),
            pl.BlockSpec((BN, D), lambda i: (i, 0)),
            pl.BlockSpec((BN, D), lambda i: (N // BN + i, 0)),
            pl.BlockSpec((D, ATT), lambda i: (0, 0)),
            pl.BlockSpec((D, ATT), lambda i: (0, 0)),
            pl.BlockSpec((2 * ATT, 1), lambda i: (0, 0)),
            pl.BlockSpec((1, D), lambda i: (0, 0)),
            pl.BlockSpec((D, D), lambda i: (0, 0)),
            pl.BlockSpec((D, D), lambda i: (0, 0)),
        ],
        out_specs=[
            pl.BlockSpec((BN, 2), lambda i: (i, 0)),
            pl.BlockSpec((BN, D), lambda i: (i, 0)),
            pl.BlockSpec((BN, D), lambda i: (i, 0)),
        ],
        out_shape=[
            jax.ShapeDtypeStruct((N, 2), _f32),
            jax.ShapeDtypeStruct((N, D), _f32),
            jax.ShapeDtypeStruct((N, D), _f32),
        ],
    )(sf, nbh, nbh, wq, wk, watt, bias, w_self2, w_rel2)


def _attn2_body(sf_ref, nb0_ref, nb1_ref, wq_ref, wk_ref, watt_ref, b_ref,
                w1_ref, b1_ref, w2_ref, b2_ref, att_ref, g_ref, acc_ref):
    i = pl.program_id(0)
    sf = sf_ref[...]
    nb = nb0_ref[...] + nb1_ref[...]
    a0, a1 = _attention(sf, nb, wq_ref, wk_ref, watt_ref)
    att_ref[...] = jnp.concatenate([a0, a1], axis=1)
    h = a0 * sf + a1 * nb + b_ref[...]

    @pl.when(i == 0)
    def _():
        acc_ref[...] = jnp.zeros_like(acc_ref)

    acc_ref[...] += jnp.sum(h, axis=0, keepdims=True)

    @pl.when(i == GRID - 1)
    def _():
        hg = acc_ref[...] * (1.0 / N)
        hid = jnp.maximum(
            jnp.dot(hg, w1_ref[...], preferred_element_type=_f32)
            + b1_ref[...], 0.0)
        g_ref[...] = (jnp.dot(hid, w2_ref[...], preferred_element_type=_f32)
                      + b2_ref[...])


def _tc_c(sf, nbh, wq, wk, watt, bias, w1, b1, w2, b2):
    return pl.pallas_call(
        _attn2_body,
        grid=(GRID,),
        in_specs=[
            pl.BlockSpec((BN, D), lambda i: (i, 0)),
            pl.BlockSpec((BN, D), lambda i: (i, 0)),
            pl.BlockSpec((BN, D), lambda i: (N // BN + i, 0)),
            pl.BlockSpec((D, ATT), lambda i: (0, 0)),
            pl.BlockSpec((D, ATT), lambda i: (0, 0)),
            pl.BlockSpec((2 * ATT, 1), lambda i: (0, 0)),
            pl.BlockSpec((1, D), lambda i: (0, 0)),
            pl.BlockSpec((D, D // 2), lambda i: (0, 0)),
            pl.BlockSpec((1, D // 2), lambda i: (0, 0)),
            pl.BlockSpec((D // 2, 1), lambda i: (0, 0)),
            pl.BlockSpec((1, 1), lambda i: (0, 0)),
        ],
        out_specs=[
            pl.BlockSpec((BN, 2), lambda i: (i, 0)),
            pl.BlockSpec((1, 1), lambda i: (0, 0)),
        ],
        out_shape=[
            jax.ShapeDtypeStruct((N, 2), _f32),
            jax.ShapeDtypeStruct((1, 1), _f32),
        ],
        scratch_shapes=[pltpu.VMEM((1, D), _f32)],
    )(sf, nbh, nbh, wq, wk, watt, bias, w1, b1, w2, b2)


def kernel(x, edge_index, l1_w_self, l1_w_rel, l1_bias, l1_wq, l1_wk, l1_watt,
           l2_w_self, l2_w_rel, l2_bias, l2_wq, l2_wk, l2_watt,
           mlp_w1, mlp_b1, mlp_w2, mlp_b2):
    dst = edge_index[0]
    src = edge_index[1]

    sf1, nbpre1 = _tc_a(x, l1_w_self, l1_w_rel)
    nb1 = _spmm(nbpre1, src, dst)
    att1, sf2, nbpre2 = _tc_b(sf1, nb1, l1_wq, l1_wk, l1_watt, l1_bias,
                              l2_w_self, l2_w_rel)
    nb2 = _spmm(nbpre2, src, dst)
    att2, gemb = _tc_c(sf2, nb2, l2_wq, l2_wk, l2_watt, l2_bias,
                       mlp_w1, mlp_b1.reshape(1, D // 2),
                       mlp_w2, mlp_b2.reshape(1, 1))
    return (gemb, att1, att2)
